# baseline (speedup 1.0000x reference)
import functools

import jax
import jax.numpy as jnp
from jax.experimental import pallas as pl
from jax.experimental.pallas import tpu as pltpu

B = 1024
K = 100000
D = 64
MLP_DIM = 512
H = 4
TOPK = 32

KPAD = 100096
CCHUNK = 4352
NCHUNK = KPAD // CCHUNK
NEG = -1e30


def _span_body(x_ref, w1_ref, b1_ref, w2_ref, b2_ref, out_ref):
    hid = jnp.maximum(
        jnp.dot(x_ref[...], w1_ref[0], preferred_element_type=jnp.float32)
        + b1_ref[0], 0.0)
    out_ref[0] = (jnp.dot(hid, w2_ref[0], preferred_element_type=jnp.float32)
                  + b2_ref[0])


def _scores_body(span_ref, kge_ref, out_ref):
    j = pl.program_id(1)
    s = jnp.dot(span_ref[0], kge_ref[...].T, preferred_element_type=jnp.float32)
    col = jax.lax.broadcasted_iota(jnp.int32, (B, CCHUNK), 1) + j * CCHUNK
    out_ref[0] = jnp.where(col < K, s, NEG)


def _final_body(cat_ref, w_ref, b_ref, out_ref):
    out_ref[...] = (jnp.dot(cat_ref[...], w_ref[...],
                            preferred_element_type=jnp.float32)
                    + b_ref[0][None, :])


def kernel(x, kge_emb, W1, b1, W2, b2, lin_W, lin_b, k):
    del k
    spanned = pl.pallas_call(
        _span_body,
        grid=(H,),
        in_specs=[
            pl.BlockSpec((B, MLP_DIM), lambda h: (0, 0)),
            pl.BlockSpec((1, MLP_DIM, MLP_DIM), lambda h: (h, 0, 0)),
            pl.BlockSpec((1, 1, MLP_DIM), lambda h: (h, 0, 0)),
            pl.BlockSpec((1, MLP_DIM, D), lambda h: (h, 0, 0)),
            pl.BlockSpec((1, 1, D), lambda h: (h, 0, 0)),
        ],
        out_specs=pl.BlockSpec((1, B, D), lambda h: (h, 0, 0)),
        out_shape=jax.ShapeDtypeStruct((H, B, D), jnp.float32),
    )(x, W1, b1.reshape(H, 1, MLP_DIM), W2, b2.reshape(H, 1, D))

    kge_pad = jnp.pad(kge_emb, ((0, KPAD - K), (0, 0)))
    scores = pl.pallas_call(
        _scores_body,
        grid=(H, NCHUNK),
        in_specs=[
            pl.BlockSpec((1, B, D), lambda h, j: (h, 0, 0)),
            pl.BlockSpec((CCHUNK, D), lambda h, j: (j, 0)),
        ],
        out_specs=pl.BlockSpec((1, B, CCHUNK), lambda h, j: (h, 0, j)),
        out_shape=jax.ShapeDtypeStruct((H, B, KPAD), jnp.float32),
    )(spanned, kge_pad)

    topmem, topmem_idx = jax.lax.top_k(scores, TOPK)
    e = jnp.exp(topmem)
    w = jax.nn.softmax(e, axis=-1)
    topval = kge_emb[topmem_idx]
    value = jnp.einsum('hbk,hbkd->hbd', w, topval)
    cat = jnp.transpose(value, (1, 0, 2)).reshape(B, H * D)

    out = pl.pallas_call(
        _final_body,
        in_specs=[
            pl.BlockSpec((B, H * D), lambda: (0, 0)),
            pl.BlockSpec((H * D, D), lambda: (0, 0)),
            pl.BlockSpec((1, D), lambda: (0, 0)),
        ],
        out_specs=pl.BlockSpec((B, D), lambda: (0, 0)),
        out_shape=jax.ShapeDtypeStruct((B, D), jnp.float32),
    )(cat, lin_W, lin_b.reshape(1, D))
    return out


# speedup vs baseline: 39.8719x; 39.8719x over previous
import functools

import jax
import jax.numpy as jnp
from jax import lax
from jax.experimental import pallas as pl
from jax.experimental.pallas import tpu as pltpu
from jax.experimental.pallas import tpu_sc as plsc

B = 1024
K = 100000
D = 64
MLP_DIM = 512
H = 4
TOPK = 32

KPAD = 102400
CCHUNK = 4096
NCHUNK = KPAD // CCHUNK
NEG = -1e30

NW = 32
ROWS_PER_W = (H * B) // NW
NV = KPAD // 16
NVH = NV // 2
UNROLL = 8
SURV_CAP = 2048
NEGF = -3e38
BIGI = (1 << 30)


def _span_body(x_ref, w1_ref, b1_ref, w2_ref, b2_ref, out_ref):
    hid = jnp.maximum(
        jnp.dot(x_ref[...], w1_ref[0], preferred_element_type=jnp.float32)
        + b1_ref[0], 0.0)
    out_ref[0] = (jnp.dot(hid, w2_ref[0], preferred_element_type=jnp.float32)
                  + b2_ref[0])


def _scores_body(span_ref, kge_ref, out_ref):
    j = pl.program_id(1)
    s = jnp.dot(span_ref[0], kge_ref[...].T, preferred_element_type=jnp.float32)
    col = jax.lax.broadcasted_iota(jnp.int32, (B, CCHUNK), 1) + j * CCHUNK
    out_ref[0] = jnp.where(col < K, s, NEG)


def _final_body(cat_ref, w_ref, b_ref, out_ref):
    out_ref[...] = (jnp.dot(cat_ref[...], w_ref[...],
                            preferred_element_type=jnp.float32)
                    + b_ref[0][None, :])


def _bmax(v):
    idx = lax.iota(jnp.int32, 16)
    for s in (8, 4, 2, 1):
        v = jnp.maximum(v, v[idx ^ s])
    return v


def _bsum(v):
    idx = lax.iota(jnp.int32, 16)
    for s in (8, 4, 2, 1):
        v = v + v[idx ^ s]
    return v


def _sc_topk_body(scores_hbm, kge_hbm, out_hbm,
                  buf, sval, sidx, tidx, rows, outv, dma_sem):
    wid = lax.axis_index("s") * 2 + lax.axis_index("c")
    row0 = wid * ROWS_PER_W

    def do_row(r, _):
        row = row0 + r
        h = row // B
        b = row - h * B
        pltpu.sync_copy(scores_hbm.at[row], buf)

        def amax(i, g):
            base = i * UNROLL * 16
            g = list(g)
            for j in range(UNROLL):
                g[j % 4] = jnp.maximum(g[j % 4], buf[pl.ds(base + j * 16, 16)])
            return tuple(g)
        ginit = jnp.full((16,), NEGF, jnp.float32)
        gs = lax.fori_loop(0, NV // UNROLL, amax,
                           (ginit, ginit, ginit, ginit))

        def tpick(p, carry):
            lastv, g0, g1, g2, g3 = carry
            cs = [jnp.where(g < lastv, g, NEGF) for g in (g0, g1, g2, g3)]
            c = jnp.maximum(jnp.maximum(cs[0], cs[1]),
                            jnp.maximum(cs[2], cs[3]))
            m = _bmax(c)[0]
            return m, g0, g1, g2, g3
        t0 = lax.fori_loop(0, TOPK, tpick,
                           (jnp.float32(3e38),) + gs)[0]

        def bfilt(i, off):
            base = i * UNROLL * 16
            vs = [buf[pl.ds(base + j * 16, 16)] for j in range(UNROLL)]
            gmax = vs[0]
            for j in range(1, UNROLL):
                gmax = jnp.maximum(gmax, vs[j])
            anyv = _bmax(gmax)[0] >= t0

            def slow(o):
                for j in range(UNROLL):
                    mj = vs[j] >= t0
                    hitj = _bmax(vs[j])[0] >= t0
                    iv = lax.iota(jnp.int32, 16) + (base + j * 16)

                    def hit(oo):
                        sval[pl.ds(oo, 16)] = jnp.where(mj, vs[j], NEGF)
                        sidx[pl.ds(oo, 16)] = jnp.where(mj, iv, BIGI)
                        return jnp.minimum(oo + 16, SURV_CAP)
                    o = lax.cond(hitj, hit, lambda oo: oo, o)
                return o
            return lax.cond(anyv, slow, lambda o: o, off)
        off = lax.fori_loop(0, NV // UNROLL, bfilt, jnp.int32(0))
        nv = off // 16

        lane = lax.iota(jnp.int32, 16)

        def pick(p, carry):
            lastv, lasti, tv1, tv2, ti1, ti2 = carry

            def scan(j, c):
                bv, bi = c
                sv = sval[pl.ds(j * 16, 16)]
                si = sidx[pl.ds(j * 16, 16)]
                elig = (sv < lastv) | ((sv == lastv) & (si > lasti))
                cv = jnp.where(elig, sv, NEGF)
                ci = jnp.where(elig, si, BIGI)
                better = (cv > bv) | ((cv == bv) & (ci < bi))
                return jnp.where(better, cv, bv), jnp.where(better, ci, bi)
            bv, bi = lax.fori_loop(
                0, nv, scan,
                (jnp.full((16,), NEGF, jnp.float32),
                 jnp.full((16,), BIGI, jnp.int32)))
            m = _bmax(bv)[0]
            mi = -(_bmax(jnp.where(bv == m, -bi, -BIGI))[0])
            in1 = lane == p
            in2 = lane == (p - 16)
            tv1 = jnp.where(in1, m, tv1)
            ti1 = jnp.where(in1, mi, ti1)
            tv2 = jnp.where(in2, m, tv2)
            ti2 = jnp.where(in2, mi, ti2)
            return m, mi, tv1, tv2, ti1, ti2
        zf = jnp.zeros((16,), jnp.float32)
        zi = jnp.zeros((16,), jnp.int32)
        _, _, tv1, tv2, ti1, ti2 = lax.fori_loop(
            0, TOPK, pick, (jnp.float32(3e38), jnp.int32(-1), zf, zf, zi, zi))
        tidx[pl.ds(0, 16)] = ti1
        tidx[pl.ds(16, 16)] = ti2

        e1 = jnp.exp(tv1)
        e2 = jnp.exp(tv2)
        em = _bmax(jnp.maximum(e1, e2))[0]
        w1 = jnp.exp(e1 - em)
        w2 = jnp.exp(e2 - em)
        ssum = _bsum(w1 + w2)
        w1 = w1 / ssum
        w2 = w2 / ssum

        pltpu.async_copy(kge_hbm.at[tidx], rows, dma_sem).wait()

        accs = [jnp.zeros((16,), jnp.float32) for _ in range(4)]
        for kk in range(TOPK):
            wk = w1[kk] if kk < 16 else w2[kk - 16]
            rk = rows.at[kk]
            accs = [a + wk * rk[pl.ds(j * 16, 16)]
                    for j, a in enumerate(accs)]
        for j in range(4):
            outv[pl.ds(j * 16, 16)] = accs[j]
        pltpu.sync_copy(outv, out_hbm.at[b, h])
        return 0
    lax.fori_loop(0, ROWS_PER_W, do_row, 0)


def kernel(x, kge_emb, W1, b1, W2, b2, lin_W, lin_b, k):
    del k
    spanned = pl.pallas_call(
        _span_body,
        grid=(H,),
        in_specs=[
            pl.BlockSpec((B, MLP_DIM), lambda h: (0, 0)),
            pl.BlockSpec((1, MLP_DIM, MLP_DIM), lambda h: (h, 0, 0)),
            pl.BlockSpec((1, 1, MLP_DIM), lambda h: (h, 0, 0)),
            pl.BlockSpec((1, MLP_DIM, D), lambda h: (h, 0, 0)),
            pl.BlockSpec((1, 1, D), lambda h: (h, 0, 0)),
        ],
        out_specs=pl.BlockSpec((1, B, D), lambda h: (h, 0, 0)),
        out_shape=jax.ShapeDtypeStruct((H, B, D), jnp.float32),
    )(x, W1, b1.reshape(H, 1, MLP_DIM), W2, b2.reshape(H, 1, D))

    kge_pad = jnp.pad(kge_emb, ((0, KPAD - K), (0, 0)))
    scores = pl.pallas_call(
        _scores_body,
        grid=(H, NCHUNK),
        in_specs=[
            pl.BlockSpec((1, B, D), lambda h, j: (h, 0, 0)),
            pl.BlockSpec((CCHUNK, D), lambda h, j: (j, 0)),
        ],
        out_specs=pl.BlockSpec((1, B, CCHUNK), lambda h, j: (h, 0, j)),
        out_shape=jax.ShapeDtypeStruct((H, B, KPAD), jnp.float32),
    )(spanned, kge_pad)

    mesh = plsc.VectorSubcoreMesh(core_axis_name="c", subcore_axis_name="s")
    value = pl.kernel(
        _sc_topk_body,
        out_type=jax.ShapeDtypeStruct((B, H, D), jnp.float32),
        mesh=mesh,
        scratch_types=[
            pltpu.VMEM((KPAD,), jnp.float32),
            pltpu.VMEM((SURV_CAP + 32,), jnp.float32),
            pltpu.VMEM((SURV_CAP + 32,), jnp.int32),
            pltpu.VMEM((TOPK,), jnp.int32),
            pltpu.VMEM((TOPK, 2 * D), jnp.float32),
            pltpu.VMEM((D,), jnp.float32),
            pltpu.SemaphoreType.DMA,
        ],
    )(scores.reshape(H * B, KPAD), jnp.pad(kge_emb, ((0, 0), (0, D))))

    out = pl.pallas_call(
        _final_body,
        in_specs=[
            pl.BlockSpec((B, H * D), lambda: (0, 0)),
            pl.BlockSpec((H * D, D), lambda: (0, 0)),
            pl.BlockSpec((1, D), lambda: (0, 0)),
        ],
        out_specs=pl.BlockSpec((B, D), lambda: (0, 0)),
        out_shape=jax.ShapeDtypeStruct((B, D), jnp.float32),
    )(value.reshape(B, H * D), lin_W, lin_b.reshape(1, D))
    return out


# speedup vs baseline: 51.6671x; 1.2958x over previous
import functools

import jax
import jax.numpy as jnp
from jax import lax
from jax.experimental import pallas as pl
from jax.experimental.pallas import tpu as pltpu
from jax.experimental.pallas import tpu_sc as plsc

B = 1024
K = 100000
D = 64
MLP_DIM = 512
H = 4
TOPK = 32

KPAD = 102400
CCHUNK = 4096
NCHUNK = KPAD // CCHUNK
NEG = -1e30

NW = 32
ROWS_PER_W = (H * B) // NW
NV = KPAD // 16
NVH = NV // 2
UNROLL = 16
NSTRIDE = 8
SURV_CAP = 2048
NEGF = -3e38
BIGI = (1 << 30)


def _span_body(x_ref, w1_ref, b1_ref, w2_ref, b2_ref, out_ref):
    hid = jnp.maximum(
        jnp.dot(x_ref[...], w1_ref[0], preferred_element_type=jnp.float32)
        + b1_ref[0], 0.0)
    out_ref[0] = (jnp.dot(hid, w2_ref[0], preferred_element_type=jnp.float32)
                  + b2_ref[0])


def _scores_body(span_ref, kge_ref, out_ref):
    j = pl.program_id(1)
    s = jnp.dot(span_ref[0], kge_ref[...].T, preferred_element_type=jnp.float32)
    col = jax.lax.broadcasted_iota(jnp.int32, (B, CCHUNK), 1) + j * CCHUNK
    out_ref[0] = jnp.where(col < K, s, NEG)


def _final_body(cat_ref, w_ref, b_ref, out_ref):
    out_ref[...] = (jnp.dot(cat_ref[...], w_ref[...],
                            preferred_element_type=jnp.float32)
                    + b_ref[0][None, :])


def _bmax(v):
    idx = lax.iota(jnp.int32, 16)
    for s in (8, 4, 2, 1):
        v = jnp.maximum(v, v[idx ^ s])
    return v


def _bsum(v):
    idx = lax.iota(jnp.int32, 16)
    for s in (8, 4, 2, 1):
        v = v + v[idx ^ s]
    return v


def _sc_topk_body(scores_hbm, kge_hbm, out_hbm,
                  buf, sval, sidx, tidx, rows, outv, dma_sem):
    wid = lax.axis_index("s") * 2 + lax.axis_index("c")
    row0 = wid * ROWS_PER_W

    def do_row(r, _):
        row = row0 + r
        h = row // B
        b = row - h * B
        pltpu.sync_copy(scores_hbm.at[row], buf)

        def amax(i, g):
            base = i * UNROLL * 16
            g = list(g)
            for j in range(UNROLL):
                g[j % NSTRIDE] = jnp.maximum(g[j % NSTRIDE],
                                             buf[pl.ds(base + j * 16, 16)])
            return tuple(g)
        ginit = jnp.full((16,), NEGF, jnp.float32)
        gs = lax.fori_loop(0, NV // UNROLL, amax, (ginit,) * NSTRIDE)

        def tpick(p, carry):
            lastv = carry[0]
            g = carry[1:]
            cs = [jnp.where(gg < lastv, gg, NEGF) for gg in g]
            c = cs[0]
            for gg in cs[1:]:
                c = jnp.maximum(c, gg)
            m = _bmax(c)[0]
            return (m,) + g
        t0 = lax.fori_loop(0, TOPK, tpick,
                           (jnp.float32(3e38),) + gs)[0]

        def bfilt(i, off):
            base = i * UNROLL * 16
            vs = [buf[pl.ds(base + j * 16, 16)] for j in range(UNROLL)]
            gmax = vs[0]
            for j in range(1, UNROLL):
                gmax = jnp.maximum(gmax, vs[j])
            anyv = _bmax(gmax)[0] >= t0

            def slow(o):
                for j in range(UNROLL):
                    mj = vs[j] >= t0
                    hitj = _bmax(vs[j])[0] >= t0
                    iv = lax.iota(jnp.int32, 16) + (base + j * 16)

                    def hit(oo):
                        sval[pl.ds(oo, 16)] = jnp.where(mj, vs[j], NEGF)
                        sidx[pl.ds(oo, 16)] = jnp.where(mj, iv, BIGI)
                        return jnp.minimum(oo + 16, SURV_CAP)
                    o = lax.cond(hitj, hit, lambda oo: oo, o)
                return o
            return lax.cond(anyv, slow, lambda o: o, off)
        off = lax.fori_loop(0, NV // UNROLL, bfilt, jnp.int32(0))
        nv = off // 16

        lane = lax.iota(jnp.int32, 16)

        def pick(p, carry):
            lastv, lasti, tv1, tv2, ti1, ti2 = carry

            def scan(j, c):
                bv, bi = c
                sv = sval[pl.ds(j * 16, 16)]
                si = sidx[pl.ds(j * 16, 16)]
                elig = (sv < lastv) | ((sv == lastv) & (si > lasti))
                cv = jnp.where(elig, sv, NEGF)
                ci = jnp.where(elig, si, BIGI)
                better = (cv > bv) | ((cv == bv) & (ci < bi))
                return jnp.where(better, cv, bv), jnp.where(better, ci, bi)
            bv, bi = lax.fori_loop(
                0, nv, scan,
                (jnp.full((16,), NEGF, jnp.float32),
                 jnp.full((16,), BIGI, jnp.int32)))
            m = _bmax(bv)[0]
            mi = -(_bmax(jnp.where(bv == m, -bi, -BIGI))[0])
            in1 = lane == p
            in2 = lane == (p - 16)
            tv1 = jnp.where(in1, m, tv1)
            ti1 = jnp.where(in1, mi, ti1)
            tv2 = jnp.where(in2, m, tv2)
            ti2 = jnp.where(in2, mi, ti2)
            return m, mi, tv1, tv2, ti1, ti2
        zf = jnp.zeros((16,), jnp.float32)
        zi = jnp.zeros((16,), jnp.int32)
        _, _, tv1, tv2, ti1, ti2 = lax.fori_loop(
            0, TOPK, pick, (jnp.float32(3e38), jnp.int32(-1), zf, zf, zi, zi))
        tidx[pl.ds(0, 16)] = ti1
        tidx[pl.ds(16, 16)] = ti2

        e1 = jnp.exp(tv1)
        e2 = jnp.exp(tv2)
        em = _bmax(jnp.maximum(e1, e2))[0]
        w1 = jnp.exp(e1 - em)
        w2 = jnp.exp(e2 - em)
        ssum = _bsum(w1 + w2)
        w1 = w1 / ssum
        w2 = w2 / ssum

        pltpu.async_copy(kge_hbm.at[tidx], rows, dma_sem).wait()

        accs = [jnp.zeros((16,), jnp.float32) for _ in range(4)]
        for kk in range(TOPK):
            wk = w1[kk] if kk < 16 else w2[kk - 16]
            rk = rows.at[kk]
            accs = [a + wk * rk[pl.ds(j * 16, 16)]
                    for j, a in enumerate(accs)]
        for j in range(4):
            outv[pl.ds(j * 16, 16)] = accs[j]
        pltpu.sync_copy(outv, out_hbm.at[b, h])
        return 0
    lax.fori_loop(0, ROWS_PER_W, do_row, 0)


def kernel(x, kge_emb, W1, b1, W2, b2, lin_W, lin_b, k):
    del k
    spanned = pl.pallas_call(
        _span_body,
        grid=(H,),
        in_specs=[
            pl.BlockSpec((B, MLP_DIM), lambda h: (0, 0)),
            pl.BlockSpec((1, MLP_DIM, MLP_DIM), lambda h: (h, 0, 0)),
            pl.BlockSpec((1, 1, MLP_DIM), lambda h: (h, 0, 0)),
            pl.BlockSpec((1, MLP_DIM, D), lambda h: (h, 0, 0)),
            pl.BlockSpec((1, 1, D), lambda h: (h, 0, 0)),
        ],
        out_specs=pl.BlockSpec((1, B, D), lambda h: (h, 0, 0)),
        out_shape=jax.ShapeDtypeStruct((H, B, D), jnp.float32),
    )(x, W1, b1.reshape(H, 1, MLP_DIM), W2, b2.reshape(H, 1, D))

    kge_pad = jnp.pad(kge_emb, ((0, KPAD - K), (0, 0)))
    scores = pl.pallas_call(
        _scores_body,
        grid=(H, NCHUNK),
        in_specs=[
            pl.BlockSpec((1, B, D), lambda h, j: (h, 0, 0)),
            pl.BlockSpec((CCHUNK, D), lambda h, j: (j, 0)),
        ],
        out_specs=pl.BlockSpec((1, B, CCHUNK), lambda h, j: (h, 0, j)),
        out_shape=jax.ShapeDtypeStruct((H, B, KPAD), jnp.float32),
    )(spanned, kge_pad)

    mesh = plsc.VectorSubcoreMesh(core_axis_name="c", subcore_axis_name="s")
    value = pl.kernel(
        _sc_topk_body,
        out_type=jax.ShapeDtypeStruct((B, H, D), jnp.float32),
        mesh=mesh,
        scratch_types=[
            pltpu.VMEM((KPAD,), jnp.float32),
            pltpu.VMEM((SURV_CAP + 32,), jnp.float32),
            pltpu.VMEM((SURV_CAP + 32,), jnp.int32),
            pltpu.VMEM((TOPK,), jnp.int32),
            pltpu.VMEM((TOPK, 2 * D), jnp.float32),
            pltpu.VMEM((D,), jnp.float32),
            pltpu.SemaphoreType.DMA,
        ],
    )(scores.reshape(H * B, KPAD), jnp.pad(kge_emb, ((0, 0), (0, D))))

    out = pl.pallas_call(
        _final_body,
        in_specs=[
            pl.BlockSpec((B, H * D), lambda: (0, 0)),
            pl.BlockSpec((H * D, D), lambda: (0, 0)),
            pl.BlockSpec((1, D), lambda: (0, 0)),
        ],
        out_specs=pl.BlockSpec((B, D), lambda: (0, 0)),
        out_shape=jax.ShapeDtypeStruct((B, D), jnp.float32),
    )(value.reshape(B, H * D), lin_W, lin_b.reshape(1, D))
    return out
